# baseline (speedup 1.0000x reference)
import jax
import jax.numpy as jnp
from jax.experimental import pallas as pl
from jax.experimental.pallas import tpu as pltpu

_N_PAPER = 100000
_N_AUTHOR = 50000
_N_FIELD = 10000
_EMBED = 128
_TOTAL = _N_PAPER + _N_AUTHOR + _N_FIELD
_CH = 10000
_NCH = _TOTAL // _CH
_NBUF = 8
_LOOKAHEAD = 6


def _src_for_chunk(c, p_ref, a_ref, f_ref):
    row = c * _CH
    if row < _N_PAPER:
        return p_ref, row
    if row < _N_PAPER + _N_AUTHOR:
        return a_ref, row - _N_PAPER
    return f_ref, row - _N_PAPER - _N_AUTHOR


def _concat_kernel(p_ref, a_ref, f_ref, o_ref, bufs, sin, son):
    in_cp = [None] * _NBUF
    out_cp = [None] * _NBUF

    def make_in(c):
        b = c % _NBUF
        src, off = _src_for_chunk(c, p_ref, a_ref, f_ref)
        cp = pltpu.make_async_copy(
            src.at[pl.ds(off, _CH)], bufs.at[b], sin.at[b])
        cp.start()
        in_cp[b] = cp

    for c in range(min(_LOOKAHEAD, _NCH)):
        make_in(c)
    for c in range(_NCH):
        b = c % _NBUF
        in_cp[b].wait()
        ocp = pltpu.make_async_copy(
            bufs.at[b], o_ref.at[pl.ds(c * _CH, _CH)], son.at[b])
        ocp.start()
        out_cp[b] = ocp
        n = c + _LOOKAHEAD
        if n < _NCH:
            nb = n % _NBUF
            if out_cp[nb] is not None:
                out_cp[nb].wait()
                out_cp[nb] = None
            make_in(n)
    for b in range(_NBUF):
        if out_cp[b] is not None:
            out_cp[b].wait()


def kernel(embed_paper, embed_author, embed_field):
    return pl.pallas_call(
        _concat_kernel,
        out_shape=jax.ShapeDtypeStruct((_TOTAL, _EMBED), jnp.float32),
        in_specs=[
            pl.BlockSpec(memory_space=pl.ANY),
            pl.BlockSpec(memory_space=pl.ANY),
            pl.BlockSpec(memory_space=pl.ANY),
        ],
        out_specs=pl.BlockSpec(memory_space=pl.ANY),
        scratch_shapes=[
            pltpu.VMEM((_NBUF, _CH, _EMBED), jnp.float32),
            pltpu.SemaphoreType.DMA((_NBUF,)),
            pltpu.SemaphoreType.DMA((_NBUF,)),
        ],
    )(embed_paper, embed_author, embed_field)


# speedup vs baseline: 1.0030x; 1.0030x over previous
import jax
import jax.numpy as jnp
from jax.experimental import pallas as pl
from jax.experimental.pallas import tpu as pltpu

_N_PAPER = 100000
_N_AUTHOR = 50000
_N_FIELD = 10000
_EMBED = 128
_TOTAL = _N_PAPER + _N_AUTHOR + _N_FIELD
_CH = 5000
_NCH = _TOTAL // _CH
_NBUF = 8
_LOOKAHEAD = 6


def _src_for_chunk(c, p_ref, a_ref, f_ref):
    row = c * _CH
    if row < _N_PAPER:
        return p_ref, row
    if row < _N_PAPER + _N_AUTHOR:
        return a_ref, row - _N_PAPER
    return f_ref, row - _N_PAPER - _N_AUTHOR


def _concat_kernel(p_ref, a_ref, f_ref, o_ref, bufs, sin, son):
    in_cp = [None] * _NBUF
    out_cp = [None] * _NBUF

    def make_in(c):
        b = c % _NBUF
        src, off = _src_for_chunk(c, p_ref, a_ref, f_ref)
        cp = pltpu.make_async_copy(
            src.at[pl.ds(off, _CH)], bufs.at[b], sin.at[b])
        cp.start()
        in_cp[b] = cp

    for c in range(min(_LOOKAHEAD, _NCH)):
        make_in(c)
    for c in range(_NCH):
        b = c % _NBUF
        in_cp[b].wait()
        ocp = pltpu.make_async_copy(
            bufs.at[b], o_ref.at[pl.ds(c * _CH, _CH)], son.at[b])
        ocp.start()
        out_cp[b] = ocp
        n = c + _LOOKAHEAD
        if n < _NCH:
            nb = n % _NBUF
            if out_cp[nb] is not None:
                out_cp[nb].wait()
                out_cp[nb] = None
            make_in(n)
    for b in range(_NBUF):
        if out_cp[b] is not None:
            out_cp[b].wait()


def kernel(embed_paper, embed_author, embed_field):
    return pl.pallas_call(
        _concat_kernel,
        out_shape=jax.ShapeDtypeStruct((_TOTAL, _EMBED), jnp.float32),
        in_specs=[
            pl.BlockSpec(memory_space=pl.ANY),
            pl.BlockSpec(memory_space=pl.ANY),
            pl.BlockSpec(memory_space=pl.ANY),
        ],
        out_specs=pl.BlockSpec(memory_space=pl.ANY),
        scratch_shapes=[
            pltpu.VMEM((_NBUF, _CH, _EMBED), jnp.float32),
            pltpu.SemaphoreType.DMA((_NBUF,)),
            pltpu.SemaphoreType.DMA((_NBUF,)),
        ],
    )(embed_paper, embed_author, embed_field)
